# Initial kernel scaffold; baseline (speedup 1.0000x reference)
import jax
import jax.numpy as jnp
from jax.experimental import pallas as pl


def kernel(tokens_list, vocab_table):
    raise NotImplementedError("write your pallas kernel here")


# speedup vs baseline: 23.6506x; 23.6506x over previous
import functools

import jax
import jax.numpy as jnp
from jax import lax
from jax.experimental import pallas as pl
from jax.experimental.pallas import tpu as pltpu
from jax.experimental.pallas import tpu_sc as plsc

_B = 4096
_S = 50
_NW = 32
_PER_W = (_B * _S) // _NW
_CHUNK = 128
_NCH = _PER_W // _CHUNK

_mesh = plsc.VectorSubcoreMesh(core_axis_name="c", subcore_axis_name="s")


@functools.partial(
    pl.kernel,
    mesh=_mesh,
    out_type=jax.ShapeDtypeStruct((_NW, _NCH, _CHUNK), jnp.int32),
    scratch_types=[
        pltpu.VMEM((_NCH, _CHUNK), jnp.int32),
        pltpu.VMEM((_NCH, _CHUNK), jnp.int32),
        pltpu.SemaphoreType.DMA,
    ],
)
def _sc_gather(idx_hbm, table_hbm, out_hbm, idx_v, out_v, sem):
    wid = lax.axis_index("s") * 2 + lax.axis_index("c")
    pltpu.sync_copy(idx_hbm.at[wid], idx_v)

    def body(j, carry):
        pltpu.async_copy(table_hbm.at[idx_v.at[j]], out_v.at[j], sem).wait()
        return carry

    lax.fori_loop(0, _NCH, body, 0)
    pltpu.sync_copy(out_v, out_hbm.at[wid])


def kernel(tokens_list, vocab_table):
    idx = tokens_list.reshape(_NW, _NCH, _CHUNK)
    out = _sc_gather(idx, vocab_table)
    return out.reshape(_B, _S)


# speedup vs baseline: 41.0486x; 1.7356x over previous
import functools

import jax
import jax.numpy as jnp
from jax import lax
from jax.experimental import pallas as pl
from jax.experimental.pallas import tpu as pltpu
from jax.experimental.pallas import tpu_sc as plsc

_B = 4096
_S = 50
_NW = 32
_PER_W = (_B * _S) // _NW
_CHUNK = 128
_NCH = _PER_W // _CHUNK

_mesh = plsc.VectorSubcoreMesh(core_axis_name="c", subcore_axis_name="s")


@functools.partial(
    pl.kernel,
    mesh=_mesh,
    out_type=jax.ShapeDtypeStruct((_NW, _NCH, _CHUNK), jnp.int32),
    scratch_types=[
        pltpu.VMEM((_NCH, _CHUNK), jnp.int32),
        pltpu.VMEM((_NCH, _CHUNK), jnp.int32),
        pltpu.SemaphoreType.DMA,
    ],
)
def _sc_gather(idx_hbm, table_hbm, out_hbm, idx_v, out_v, sem):
    wid = lax.axis_index("s") * 2 + lax.axis_index("c")
    pltpu.sync_copy(idx_hbm.at[wid], idx_v)

    _K = 10
    _NG = _NCH // _K

    def fire(g):
        base = g * _K
        for i in range(_K):
            pltpu.async_copy(table_hbm.at[idx_v.at[base + i]], out_v.at[base + i], sem)

    def drain(g):
        base = g * _K
        for i in range(_K):
            pltpu.make_async_copy(
                table_hbm.at[idx_v.at[base + i]], out_v.at[base + i], sem
            ).wait()

    fire(0)

    def body(g, carry):
        fire(g + 1)
        drain(g)
        return carry

    lax.fori_loop(0, _NG - 1, body, 0)
    drain(_NG - 1)
    pltpu.sync_copy(out_v, out_hbm.at[wid])


def kernel(tokens_list, vocab_table):
    idx = tokens_list.reshape(_NW, _NCH, _CHUNK)
    out = _sc_gather(idx, vocab_table)
    return out.reshape(_B, _S)


# speedup vs baseline: 44.4626x; 1.0832x over previous
import functools

import jax
import jax.numpy as jnp
from jax import lax
from jax.experimental import pallas as pl
from jax.experimental.pallas import tpu as pltpu
from jax.experimental.pallas import tpu_sc as plsc

_B = 4096
_S = 50
_NW = 32
_PER_W = (_B * _S) // _NW
_CHUNK = 128
_NCH = _PER_W // _CHUNK

_mesh = plsc.VectorSubcoreMesh(core_axis_name="c", subcore_axis_name="s")


@functools.partial(
    pl.kernel,
    mesh=_mesh,
    out_type=jax.ShapeDtypeStruct((_NW, _PER_W), jnp.int32),
    scratch_types=[
        pltpu.VMEM((_PER_W,), jnp.int32),
        pltpu.VMEM((_PER_W,), jnp.int32),
        pltpu.SemaphoreType.DMA,
    ],
)
def _sc_gather(idx_hbm, table_hbm, out_hbm, idx_v, out_v, sem):
    wid = lax.axis_index("s") * 2 + lax.axis_index("c")
    pltpu.sync_copy(idx_hbm.at[wid], idx_v)

    pltpu.async_copy(table_hbm.at[idx_v], out_v, sem).wait()
    pltpu.sync_copy(out_v, out_hbm.at[wid])


def kernel(tokens_list, vocab_table):
    idx = tokens_list.reshape(_NW, _PER_W)
    out = _sc_gather(idx, vocab_table)
    return out.reshape(_B, _S)
